# Initial kernel scaffold; baseline (speedup 1.0000x reference)
import jax
import jax.numpy as jnp
from jax.experimental import pallas as pl


def kernel(raw, beautified, wm0, b0, wm1, b1, wm2, b2, wm3, b3, wm4, b4):
    raise NotImplementedError("write your pallas kernel here")


# speedup vs baseline: 11.1183x; 11.1183x over previous
import functools

import jax
import jax.numpy as jnp
from jax.experimental import pallas as pl
from jax.experimental.pallas import tpu as pltpu


def _space_to_depth_pad(x):
    N, H, W, C = x.shape
    x = jnp.pad(x, ((0, 0), (1, 1), (1, 1), (0, 0)))
    Hs, Ws = (H + 2) // 2, (W + 2) // 2
    x = x.reshape(N, Hs, 2, Ws, 2, C)
    x = x.transpose(0, 1, 3, 2, 4, 5)
    return x.reshape(N, Hs, Ws, 4 * C)


def _tap_weights(w):
    _, _, cin, cout = w.shape
    w = w.reshape(2, 2, 2, 2, cin, cout)
    w = w.transpose(0, 2, 1, 3, 4, 5)
    return w.reshape(2, 2, 4 * cin, cout)


def _conv_layer_kernel(x_ref, w_ref, b_ref, o_ref, *, inorm, eps, slope):
    x = x_ref[0]
    kc = x.shape[-1]
    _, ho, wo, cout = o_ref.shape
    m = ho * wo
    acc = jnp.zeros((m, cout), jnp.float32)
    for di in range(2):
        for dj in range(2):
            a = x[di:di + ho, dj:dj + wo, :].reshape(m, kc)
            acc = acc + jnp.dot(a, w_ref[di, dj],
                                preferred_element_type=jnp.float32)
    acc = acc + b_ref[...].astype(jnp.float32)
    if inorm:
        mean = jnp.mean(acc, axis=0, keepdims=True)
        cen = acc - mean
        var = jnp.mean(cen * cen, axis=0, keepdims=True)
        acc = cen * jax.lax.rsqrt(var + eps)
    acc = jnp.where(acc >= 0.0, acc, acc * slope)
    o_ref[0] = acc.reshape(ho, wo, cout).astype(o_ref.dtype)


def _conv_layer(xs, w4, b, *, inorm):
    N, Hs, Ws, Kc = xs.shape
    cout = w4.shape[-1]
    ho, wo = Hs - 1, Ws - 1
    fn = functools.partial(_conv_layer_kernel, inorm=inorm, eps=1e-5, slope=0.2)
    return pl.pallas_call(
        fn,
        out_shape=jax.ShapeDtypeStruct((N, ho, wo, cout), jnp.bfloat16),
        grid=(N,),
        in_specs=[
            pl.BlockSpec((1, Hs, Ws, Kc), lambda n: (n, 0, 0, 0)),
            pl.BlockSpec((2, 2, Kc, cout), lambda n: (0, 0, 0, 0)),
            pl.BlockSpec((1, cout), lambda n: (0, 0)),
        ],
        out_specs=pl.BlockSpec((1, ho, wo, cout), lambda n: (n, 0, 0, 0)),
        compiler_params=pltpu.CompilerParams(dimension_semantics=("parallel",)),
    )(xs, w4, b.reshape(1, cout))


def _final_kernel(x_ref, w_ref, b_ref, o_ref):
    x = x_ref[0].astype(jnp.float32)
    _, ho, wo = o_ref.shape
    acc = jnp.zeros((ho, wo), jnp.float32)
    for kh in range(4):
        for kw in range(4):
            a = x[kh:kh + ho, kw:kw + wo, :]
            wt = w_ref[kh * 4 + kw].astype(jnp.float32)
            acc = acc + jnp.sum(a * wt, axis=-1)
    o_ref[0] = acc + b_ref[0, 0]


def _final_layer(x, w16, b):
    x = jnp.pad(x, ((0, 0), (1, 1), (1, 1), (0, 0)))
    N, Hp, Wp, C = x.shape
    ho, wo = Hp - 3, Wp - 3
    return pl.pallas_call(
        _final_kernel,
        out_shape=jax.ShapeDtypeStruct((N, ho, wo), jnp.float32),
        grid=(N,),
        in_specs=[
            pl.BlockSpec((1, Hp, Wp, C), lambda n: (n, 0, 0, 0)),
            pl.BlockSpec((16, C), lambda n: (0, 0)),
            pl.BlockSpec((1, 1), lambda n: (0, 0)),
        ],
        out_specs=pl.BlockSpec((1, ho, wo), lambda n: (n, 0, 0)),
        compiler_params=pltpu.CompilerParams(dimension_semantics=("parallel",)),
    )(x, w16, b.reshape(1, 1))


def kernel(raw, beautified, wm0, b0, wm1, b1, wm2, b2, wm3, b3, wm4, b4):
    N = raw.shape[0]
    x = jnp.concatenate([raw, beautified], axis=1)
    x = jnp.transpose(x, (0, 2, 3, 1)).astype(jnp.bfloat16)

    w0 = _tap_weights(wm0.reshape(4, 4, 16, 64)[:, :, :12, :])
    w1 = _tap_weights(wm1.reshape(4, 4, 64, 128))
    w2 = _tap_weights(wm2.reshape(4, 4, 128, 256))
    w3 = _tap_weights(wm3.reshape(4, 4, 256, 512))
    w4 = wm4.reshape(16, 512)

    y = _conv_layer(_space_to_depth_pad(x), w0, b0, inorm=False)
    y = _conv_layer(_space_to_depth_pad(y), w1, b1, inorm=True)
    y = _conv_layer(_space_to_depth_pad(y), w2, b2, inorm=True)
    y = _conv_layer(_space_to_depth_pad(y), w3, b3, inorm=True)
    y = _final_layer(y, w4, b4)
    return y[:, None, :, :]


# speedup vs baseline: 14.1891x; 1.2762x over previous
import functools

import jax
import jax.numpy as jnp
from jax.experimental import pallas as pl
from jax.experimental.pallas import tpu as pltpu


def _space_to_depth_pad(x):
    N, H, W, C = x.shape
    x = jnp.pad(x, ((0, 0), (1, 1), (1, 1), (0, 0)))
    Hs, Ws = (H + 2) // 2, (W + 2) // 2
    x = x.reshape(N, Hs, 2, Ws, 2, C)
    x = x.transpose(0, 1, 3, 2, 4, 5)
    return x.reshape(N, Hs, Ws, 4 * C)


def _s2d_val(y):
    h, w, c = y.shape
    y = jnp.pad(y, ((1, 1), (1, 1), (0, 0)))
    hs, ws = (h + 2) // 2, (w + 2) // 2
    y = y.reshape(hs, 2, ws, 2, c)
    y = y.transpose(0, 2, 1, 3, 4)
    return y.reshape(hs, ws, 4 * c)


def _tap_weights(w):
    _, _, cin, cout = w.shape
    w = w.reshape(2, 2, 2, 2, cin, cout)
    w = w.transpose(0, 2, 1, 3, 4, 5)
    return w.reshape(2, 2, 4 * cin, cout)


def _conv_block(x, w_ref, b_ref, *, inorm, eps=1e-5, slope=0.2):
    hs, ws, kc = x.shape
    cout = w_ref.shape[-1]
    ho, wo = hs - 1, ws - 1
    m = ho * wo
    acc = jnp.zeros((m, cout), jnp.float32)
    for di in range(2):
        for dj in range(2):
            a = x[di:di + ho, dj:dj + wo, :].reshape(m, kc)
            acc = acc + jnp.dot(a, w_ref[di, dj],
                                preferred_element_type=jnp.float32)
    acc = acc + b_ref[...].astype(jnp.float32)
    if inorm:
        mean = jnp.mean(acc, axis=0, keepdims=True)
        cen = acc - mean
        var = jnp.mean(cen * cen, axis=0, keepdims=True)
        acc = cen * jax.lax.rsqrt(var + eps)
    acc = jnp.where(acc >= 0.0, acc, acc * slope)
    return acc.reshape(ho, wo, cout).astype(jnp.bfloat16)


def _net_kernel(x_ref, w0_ref, b0_ref, w1_ref, b1_ref, w2_ref, b2_ref,
                w3_ref, b3_ref, w4_ref, b4_ref, o_ref):
    x = x_ref[0]
    y = _conv_block(x, w0_ref, b0_ref, inorm=False)
    y = _conv_block(_s2d_val(y), w1_ref, b1_ref, inorm=True)
    y = _conv_block(_s2d_val(y), w2_ref, b2_ref, inorm=True)
    y = _conv_block(_s2d_val(y), w3_ref, b3_ref, inorm=True)

    yp = jnp.pad(y, ((1, 1), (1, 1), (0, 0))).astype(jnp.float32)
    ho, wo = yp.shape[0] - 3, yp.shape[1] - 3
    acc = jnp.zeros((ho, wo), jnp.float32)
    for kh in range(4):
        for kw in range(4):
            a = yp[kh:kh + ho, kw:kw + wo, :]
            wt = w4_ref[kh * 4 + kw].astype(jnp.float32)
            acc = acc + jnp.sum(a * wt, axis=-1)
    o_ref[0] = acc + b4_ref[0, 0]


def kernel(raw, beautified, wm0, b0, wm1, b1, wm2, b2, wm3, b3, wm4, b4):
    N = raw.shape[0]
    x = jnp.concatenate([raw, beautified], axis=1)
    x = jnp.transpose(x, (0, 2, 3, 1)).astype(jnp.bfloat16)
    xs = _space_to_depth_pad(x)
    _, Hs, Ws, Kc = xs.shape
    fo = raw.shape[2] // 16 - 1

    w0 = _tap_weights(wm0.reshape(4, 4, 16, 64)[:, :, :12, :])
    w1 = _tap_weights(wm1.reshape(4, 4, 64, 128))
    w2 = _tap_weights(wm2.reshape(4, 4, 128, 256))
    w3 = _tap_weights(wm3.reshape(4, 4, 256, 512))
    w4 = wm4.reshape(16, 512)

    def inv(shape):
        return pl.BlockSpec(shape, lambda n: tuple(0 for _ in shape))

    out = pl.pallas_call(
        _net_kernel,
        out_shape=jax.ShapeDtypeStruct((N, fo, fo), jnp.float32),
        grid=(N,),
        in_specs=[
            pl.BlockSpec((1, Hs, Ws, Kc), lambda n: (n, 0, 0, 0)),
            inv(w0.shape), inv((1, 64)),
            inv(w1.shape), inv((1, 128)),
            inv(w2.shape), inv((1, 256)),
            inv(w3.shape), inv((1, 512)),
            inv(w4.shape), inv((1, 1)),
        ],
        out_specs=pl.BlockSpec((1, fo, fo), lambda n: (n, 0, 0)),
        compiler_params=pltpu.CompilerParams(dimension_semantics=("parallel",)),
    )(xs, w0, b0.reshape(1, 64), w1, b1.reshape(1, 128),
      w2, b2.reshape(1, 256), w3, b3.reshape(1, 512),
      w4, b4.reshape(1, 1))
    return out[:, None, :, :]


# speedup vs baseline: 30.2312x; 2.1306x over previous
import functools

import jax
import jax.numpy as jnp
from jax.experimental import pallas as pl
from jax.experimental.pallas import tpu as pltpu


def _space_to_depth_pad(x):
    N, H, W, C = x.shape
    x = jnp.pad(x, ((0, 0), (1, 1), (1, 1), (0, 0)))
    Hs, Ws = (H + 2) // 2, (W + 2) // 2
    x = x.reshape(N, Hs, 2, Ws, 2, C)
    x = x.transpose(0, 1, 3, 2, 4, 5)
    return x.reshape(N, Hs, Ws, 4 * C)


def _s2d_val(y):
    h, w, c = y.shape
    y = jnp.pad(y, ((1, 1), (1, 1), (0, 0)))
    hs, ws = (h + 2) // 2, (w + 2) // 2
    y = y.reshape(hs, 2, ws, 2, c)
    y = y.transpose(0, 2, 1, 3, 4)
    return y.reshape(hs, ws, 4 * c)


def _tap_weights(w):
    _, _, cin, cout = w.shape
    w = w.reshape(2, 2, 2, 2, cin, cout)
    w = w.transpose(0, 2, 1, 3, 4, 5)
    return w.reshape(2, 2, 4 * cin, cout)


def _conv_block(x, w_ref, b_ref, *, inorm, eps=1e-5, slope=0.2):
    hs, ws, kc = x.shape
    cout = w_ref.shape[-1]
    ho, wo = hs - 1, ws - 1
    m = ho * wo
    acc = jnp.zeros((m, cout), jnp.float32)
    for di in range(2):
        for dj in range(2):
            a = x[di:di + ho, dj:dj + wo, :].reshape(m, kc)
            acc = acc + jnp.dot(a, w_ref[di, dj],
                                preferred_element_type=jnp.float32)
    acc = acc + b_ref[...].astype(jnp.float32)
    if inorm:
        mean = jnp.mean(acc, axis=0, keepdims=True)
        cen = acc - mean
        var = jnp.mean(cen * cen, axis=0, keepdims=True)
        acc = cen * jax.lax.rsqrt(var + eps)
    acc = jnp.where(acc >= 0.0, acc, acc * slope)
    return acc.reshape(ho, wo, cout).astype(jnp.bfloat16)


def _net_kernel(raw_ref, bea_ref, w0_ref, b0_ref, w1_ref, b1_ref, w2_ref,
                b2_ref, w3_ref, b3_ref, w4_ref, b4_ref, o_ref):
    cc, h, w = raw_ref.shape[1], raw_ref.shape[2], raw_ref.shape[3]
    x = jnp.concatenate([raw_ref[0], bea_ref[0]], axis=0)
    x = x.astype(jnp.bfloat16).reshape(2 * cc, h * w)
    x = jnp.swapaxes(x, 0, 1).reshape(h, w, 2 * cc)
    x = _s2d_val(x)
    y = _conv_block(x, w0_ref, b0_ref, inorm=False)
    y = _conv_block(_s2d_val(y), w1_ref, b1_ref, inorm=True)
    y = _conv_block(_s2d_val(y), w2_ref, b2_ref, inorm=True)
    y = _conv_block(_s2d_val(y), w3_ref, b3_ref, inorm=True)

    yp = jnp.pad(y, ((1, 1), (1, 1), (0, 0))).astype(jnp.float32)
    ho, wo = yp.shape[0] - 3, yp.shape[1] - 3
    acc = jnp.zeros((ho, wo), jnp.float32)
    for kh in range(4):
        for kw in range(4):
            a = yp[kh:kh + ho, kw:kw + wo, :]
            wt = w4_ref[kh * 4 + kw].astype(jnp.float32)
            acc = acc + jnp.sum(a * wt, axis=-1)
    o_ref[0] = acc + b4_ref[0, 0]


def kernel(raw, beautified, wm0, b0, wm1, b1, wm2, b2, wm3, b3, wm4, b4):
    N, C, H, W = raw.shape
    fo = H // 16 - 1

    w0 = _tap_weights(wm0.reshape(4, 4, 16, 64)[:, :, :12, :])
    w1 = _tap_weights(wm1.reshape(4, 4, 64, 128))
    w2 = _tap_weights(wm2.reshape(4, 4, 128, 256))
    w3 = _tap_weights(wm3.reshape(4, 4, 256, 512))
    w4 = wm4.reshape(16, 512)

    def inv(shape):
        return pl.BlockSpec(shape, lambda n: tuple(0 for _ in shape))

    out = pl.pallas_call(
        _net_kernel,
        out_shape=jax.ShapeDtypeStruct((N, fo, fo), jnp.float32),
        grid=(N,),
        in_specs=[
            pl.BlockSpec((1, C, H, W), lambda n: (n, 0, 0, 0)),
            pl.BlockSpec((1, C, H, W), lambda n: (n, 0, 0, 0)),
            inv(w0.shape), inv((1, 64)),
            inv(w1.shape), inv((1, 128)),
            inv(w2.shape), inv((1, 256)),
            inv(w3.shape), inv((1, 512)),
            inv(w4.shape), inv((1, 1)),
        ],
        out_specs=pl.BlockSpec((1, fo, fo), lambda n: (n, 0, 0)),
        compiler_params=pltpu.CompilerParams(dimension_semantics=("parallel",)),
    )(raw, beautified, w0, b0.reshape(1, 64), w1, b1.reshape(1, 128),
      w2, b2.reshape(1, 256), w3, b3.reshape(1, 512),
      w4, b4.reshape(1, 1))
    return out[:, None, :, :]


# speedup vs baseline: 30.2405x; 1.0003x over previous
import jax
import jax.numpy as jnp
from jax.experimental import pallas as pl
from jax.experimental.pallas import tpu as pltpu


def _s2d_val(y):
    h, w, c = y.shape
    y = jnp.pad(y, ((1, 1), (1, 1), (0, 0)))
    hs, ws = (h + 2) // 2, (w + 2) // 2
    y = y.reshape(hs, 2, ws, 2, c)
    y = y.transpose(0, 2, 1, 3, 4)
    return y.reshape(hs, ws, 4 * c)


def _tap_weights(w):
    _, _, cin, cout = w.shape
    w = w.reshape(2, 2, 2, 2, cin, cout)
    w = w.transpose(0, 2, 1, 3, 4, 5)
    return w.reshape(2, 2, 4 * cin, cout)


def _conv_block(x, w_ref, b_ref, *, inorm, eps=1e-5, slope=0.2):
    hs, ws, kc = x.shape
    cout = w_ref.shape[-1]
    ho, wo = hs - 1, ws - 1
    m = ho * wo
    acc = jnp.zeros((m, cout), jnp.float32)
    for di in range(2):
        for dj in range(2):
            a = x[di:di + ho, dj:dj + wo, :].reshape(m, kc)
            acc = acc + jnp.dot(a, w_ref[di, dj],
                                preferred_element_type=jnp.float32)
    acc = acc + b_ref[...].astype(jnp.float32)
    if inorm:
        mean = jnp.mean(acc, axis=0, keepdims=True)
        cen = acc - mean
        var = jnp.mean(cen * cen, axis=0, keepdims=True)
        acc = cen * jax.lax.rsqrt(var + eps)
    acc = jnp.where(acc >= 0.0, acc, acc * slope)
    return acc.reshape(ho, wo, cout).astype(jnp.bfloat16)


def _net_kernel(raw_ref, bea_ref, w0_ref, b0_ref, w1_ref, b1_ref, w2_ref,
                b2_ref, w3_ref, b3_ref, w4_ref, b4_ref, o_ref):
    cc, h, w = raw_ref.shape[1], raw_ref.shape[2], raw_ref.shape[3]
    x = jnp.concatenate([raw_ref[0], bea_ref[0]], axis=0)
    x = x.astype(jnp.bfloat16).reshape(2 * cc, h * w)
    x = jnp.swapaxes(x, 0, 1).reshape(h, w, 2 * cc)
    x = _s2d_val(x)
    y = _conv_block(x, w0_ref, b0_ref, inorm=False)
    y = _conv_block(_s2d_val(y), w1_ref, b1_ref, inorm=True)
    y = _conv_block(_s2d_val(y), w2_ref, b2_ref, inorm=True)
    y = _conv_block(_s2d_val(y), w3_ref, b3_ref, inorm=True)

    yp = jnp.pad(y, ((1, 1), (1, 1), (0, 0))).astype(jnp.float32)
    ho, wo = yp.shape[0] - 3, yp.shape[1] - 3
    acc = jnp.zeros((ho, wo), jnp.float32)
    for kh in range(4):
        for kw in range(4):
            a = yp[kh:kh + ho, kw:kw + wo, :]
            wt = w4_ref[kh * 4 + kw].astype(jnp.float32)
            acc = acc + jnp.sum(a * wt, axis=-1)
    o_ref[0] = acc + b4_ref[0, 0]


def kernel(raw, beautified, wm0, b0, wm1, b1, wm2, b2, wm3, b3, wm4, b4):
    N, C, H, W = raw.shape
    fo = H // 16 - 1

    w0 = _tap_weights(wm0.reshape(4, 4, 16, 64)[:, :, :12, :])
    w1 = _tap_weights(wm1.reshape(4, 4, 64, 128))
    w2 = _tap_weights(wm2.reshape(4, 4, 128, 256))
    w3 = _tap_weights(wm3.reshape(4, 4, 256, 512))
    w4 = wm4.reshape(16, 512)

    def inv(shape):
        ndim = len(shape)
        return pl.BlockSpec(shape, lambda a, b, _nd=ndim: (0,) * _nd)

    out = pl.pallas_call(
        _net_kernel,
        out_shape=jax.ShapeDtypeStruct((N, fo, fo), jnp.float32),
        grid=(2, N // 2),
        in_specs=[
            pl.BlockSpec((1, C, H, W), lambda a, b: (a * (N // 2) + b, 0, 0, 0)),
            pl.BlockSpec((1, C, H, W), lambda a, b: (a * (N // 2) + b, 0, 0, 0)),
            inv(w0.shape), inv((1, 64)),
            inv(w1.shape), inv((1, 128)),
            inv(w2.shape), inv((1, 256)),
            inv(w3.shape), inv((1, 512)),
            inv(w4.shape), inv((1, 1)),
        ],
        out_specs=pl.BlockSpec((1, fo, fo), lambda a, b: (a * (N // 2) + b, 0, 0)),
        compiler_params=pltpu.CompilerParams(
            dimension_semantics=("parallel", "parallel")),
    )(raw, beautified, w0, b0.reshape(1, 64), w1, b1.reshape(1, 128),
      w2, b2.reshape(1, 256), w3, b3.reshape(1, 512),
      w4, b4.reshape(1, 1))
    return out[:, None, :, :]


# speedup vs baseline: 33.3734x; 1.1036x over previous
import jax
import jax.numpy as jnp
from jax.experimental import pallas as pl
from jax.experimental.pallas import tpu as pltpu


def _conv_from_scratch(s_ref, w_ref, b_ref, ho, wo, *, inorm, eps=1e-5,
                       slope=0.2):
    srefs = s_ref if isinstance(s_ref, (list, tuple)) else [s_ref]
    cg = w_ref.shape[-2] // len(srefs)
    cout = w_ref.shape[-1]
    m = ho * wo
    acc = jnp.zeros((m, cout), jnp.float32)
    for kh in range(4):
        for kw in range(4):
            for g, sr in enumerate(srefs):
                a = sr[kh:kh + 2 * ho - 1:2, kw:kw + 2 * wo - 1:2, 0:cg]
                a = a.reshape(m, cg).astype(jnp.bfloat16)
                acc = acc + jnp.dot(a, w_ref[kh, kw, g * cg:(g + 1) * cg, :],
                                    preferred_element_type=jnp.float32)
    acc = acc + b_ref[...].astype(jnp.float32)
    if inorm:
        mean = jnp.mean(acc, axis=0, keepdims=True)
        cen = acc - mean
        var = jnp.mean(cen * cen, axis=0, keepdims=True)
        acc = cen * jax.lax.rsqrt(var + eps)
    acc = jnp.where(acc >= 0.0, acc, acc * slope)
    return acc.reshape(ho, wo, cout)


def _store_padded(s_ref, y):
    h, w, c = y.shape
    s_ref[0:1, :, 0:c] = jnp.zeros((1, w + 2, c), s_ref.dtype)
    s_ref[h + 1:h + 2, :, 0:c] = jnp.zeros((1, w + 2, c), s_ref.dtype)
    s_ref[:, 0:1, 0:c] = jnp.zeros((h + 2, 1, c), s_ref.dtype)
    s_ref[:, w + 1:w + 2, 0:c] = jnp.zeros((h + 2, 1, c), s_ref.dtype)
    s_ref[1:h + 1, 1:w + 1, 0:c] = y


def _net_kernel(raw_ref, bea_ref, w0_ref, b0_ref, w1_ref, b1_ref, w2_ref,
                b2_ref, w3_ref, b3_ref, w4_ref, b4_ref, o_ref,
                s0, s1, s2, s3a, s3b):
    cc, h, w = raw_ref.shape[1], raw_ref.shape[2], raw_ref.shape[3]
    x = jnp.concatenate([raw_ref[0], bea_ref[0]], axis=0)
    x = x.reshape(2 * cc, h * w)
    x = jnp.swapaxes(x, 0, 1).reshape(h, w, 2 * cc)

    _store_padded(s0, x)
    y = _conv_from_scratch(s0, w0_ref, b0_ref, h // 2, w // 2, inorm=False)
    _store_padded(s1, y)
    y = _conv_from_scratch(s1, w1_ref, b1_ref, h // 4, w // 4, inorm=True)
    _store_padded(s2, y)
    y = _conv_from_scratch(s2, w2_ref, b2_ref, h // 8, w // 8, inorm=True)
    _store_padded(s3a, y[:, :, 0:128])
    _store_padded(s3b, y[:, :, 128:256])
    y = _conv_from_scratch([s3a, s3b], w3_ref, b3_ref, h // 16, w // 16,
                           inorm=True)

    yp = jnp.pad(y.astype(jnp.bfloat16), ((1, 1), (1, 1), (0, 0)))
    yp = yp.astype(jnp.float32)
    ho, wo = yp.shape[0] - 3, yp.shape[1] - 3
    acc = jnp.zeros((ho, wo), jnp.float32)
    for kh in range(4):
        for kw in range(4):
            a = yp[kh:kh + ho, kw:kw + wo, :]
            wt = w4_ref[kh * 4 + kw].astype(jnp.float32)
            acc = acc + jnp.sum(a * wt, axis=-1)
    o_ref[0] = acc + b4_ref[0, 0]


def kernel(raw, beautified, wm0, b0, wm1, b1, wm2, b2, wm3, b3, wm4, b4):
    N, C, H, W = raw.shape
    fo = H // 16 - 1

    w0 = wm0.reshape(4, 4, 16, 64)[:, :, :12, :]
    w1 = wm1.reshape(4, 4, 64, 128)
    w2 = wm2.reshape(4, 4, 128, 256)
    w3 = wm3.reshape(4, 4, 256, 512)
    w4 = wm4.reshape(16, 512)

    def inv(shape):
        ndim = len(shape)
        return pl.BlockSpec(shape, lambda n, _nd=ndim: (0,) * _nd)

    out = pl.pallas_call(
        _net_kernel,
        out_shape=jax.ShapeDtypeStruct((N, fo, fo), jnp.float32),
        grid=(N,),
        in_specs=[
            pl.BlockSpec((1, C, H, W), lambda n: (n, 0, 0, 0)),
            pl.BlockSpec((1, C, H, W), lambda n: (n, 0, 0, 0)),
            inv(w0.shape), inv((1, 64)),
            inv(w1.shape), inv((1, 128)),
            inv(w2.shape), inv((1, 256)),
            inv(w3.shape), inv((1, 512)),
            inv(w4.shape), inv((1, 1)),
        ],
        out_specs=pl.BlockSpec((1, fo, fo), lambda n: (n, 0, 0)),
        scratch_shapes=[
            pltpu.VMEM((H + 2, W + 2, 128), jnp.float32),
            pltpu.VMEM((H // 2 + 2, W // 2 + 2, 128), jnp.float32),
            pltpu.VMEM((H // 4 + 2, W // 4 + 2, 128), jnp.float32),
            pltpu.VMEM((H // 8 + 2, W // 8 + 2, 128), jnp.float32),
            pltpu.VMEM((H // 8 + 2, W // 8 + 2, 128), jnp.float32),
        ],
        compiler_params=pltpu.CompilerParams(dimension_semantics=("parallel",)),
    )(raw, beautified, w0, b0.reshape(1, 64), w1, b1.reshape(1, 128),
      w2, b2.reshape(1, 256), w3, b3.reshape(1, 512),
      w4, b4.reshape(1, 1))
    return out[:, None, :, :]


# speedup vs baseline: 35.4752x; 1.0630x over previous
import jax
import jax.numpy as jnp
from jax.experimental import pallas as pl
from jax.experimental.pallas import tpu as pltpu


def _conv_from_scratch(s_ref, w_ref, b_ref, ho, wo, *, inorm, eps=1e-5,
                       slope=0.2):
    srefs = s_ref if isinstance(s_ref, (list, tuple)) else [s_ref]
    cg = w_ref.shape[-2] // len(srefs)
    cout = w_ref.shape[-1]
    m = ho * wo
    acc = jnp.zeros((m, cout), jnp.float32)
    for kh in range(4):
        for kw in range(4):
            for g, sr in enumerate(srefs):
                a = sr[kh:kh + 2 * ho - 1:2, kw:kw + 2 * wo - 1:2, 0:cg]
                a = a.reshape(m, cg).astype(jnp.bfloat16)
                acc = acc + jnp.dot(a, w_ref[kh, kw, g * cg:(g + 1) * cg, :],
                                    preferred_element_type=jnp.float32)
    acc = acc + b_ref[...].astype(jnp.float32)
    if inorm:
        mean = jnp.mean(acc, axis=0, keepdims=True)
        cen = acc - mean
        var = jnp.mean(cen * cen, axis=0, keepdims=True)
        acc = cen * jax.lax.rsqrt(var + eps)
    acc = jnp.where(acc >= 0.0, acc, acc * slope)
    return acc.reshape(ho, wo, cout)


def _zero_border(s_ref, h, w, c):
    s_ref[0:1, :, 0:c] = jnp.zeros((1, w + 2, c), s_ref.dtype)
    s_ref[h + 1:h + 2, :, 0:c] = jnp.zeros((1, w + 2, c), s_ref.dtype)
    s_ref[:, 0:1, 0:c] = jnp.zeros((h + 2, 1, c), s_ref.dtype)
    s_ref[:, w + 1:w + 2, 0:c] = jnp.zeros((h + 2, 1, c), s_ref.dtype)


def _store_padded(s_ref, y):
    h, w, c = y.shape
    s_ref[1:h + 1, 1:w + 1, 0:c] = y


def _net_kernel(raw_ref, bea_ref, w0_ref, b0_ref, w1_ref, b1_ref, w2_ref,
                b2_ref, w3_ref, b3_ref, w4_ref, b4_ref, o_ref,
                s0, s1, s2, s3a, s3b):
    cc, h, w = raw_ref.shape[1], raw_ref.shape[2], raw_ref.shape[3]

    @pl.when(pl.program_id(0) == 0)
    def _():
        _zero_border(s0, h, w, 2 * cc)
        _zero_border(s1, h // 2, w // 2, 64)
        _zero_border(s2, h // 4, w // 4, 128)
        _zero_border(s3a, h // 8, w // 8, 128)
        _zero_border(s3b, h // 8, w // 8, 128)

    x = jnp.concatenate([raw_ref[0], bea_ref[0]], axis=0)
    x = x.astype(jnp.bfloat16).reshape(2 * cc, h * w)
    x = jnp.swapaxes(x, 0, 1).reshape(h, w, 2 * cc)
    x = x.astype(jnp.float32)

    _store_padded(s0, x)
    y = _conv_from_scratch(s0, w0_ref, b0_ref, h // 2, w // 2, inorm=False)
    _store_padded(s1, y)
    y = _conv_from_scratch(s1, w1_ref, b1_ref, h // 4, w // 4, inorm=True)
    _store_padded(s2, y)
    y = _conv_from_scratch(s2, w2_ref, b2_ref, h // 8, w // 8, inorm=True)
    _store_padded(s3a, y[:, :, 0:128])
    _store_padded(s3b, y[:, :, 128:256])
    y = _conv_from_scratch([s3a, s3b], w3_ref, b3_ref, h // 16, w // 16,
                           inorm=True)

    yp = jnp.pad(y.astype(jnp.bfloat16), ((1, 1), (1, 1), (0, 0)))
    yp = yp.astype(jnp.float32)
    ho, wo = yp.shape[0] - 3, yp.shape[1] - 3
    acc = jnp.zeros((ho, wo), jnp.float32)
    for kh in range(4):
        for kw in range(4):
            a = yp[kh:kh + ho, kw:kw + wo, :]
            wt = w4_ref[kh * 4 + kw].astype(jnp.float32)
            acc = acc + jnp.sum(a * wt, axis=-1)
    o_ref[0] = acc + b4_ref[0, 0]


def kernel(raw, beautified, wm0, b0, wm1, b1, wm2, b2, wm3, b3, wm4, b4):
    N, C, H, W = raw.shape
    fo = H // 16 - 1

    w0 = wm0.reshape(4, 4, 16, 64)[:, :, :12, :]
    w1 = wm1.reshape(4, 4, 64, 128)
    w2 = wm2.reshape(4, 4, 128, 256)
    w3 = wm3.reshape(4, 4, 256, 512)
    w4 = wm4.reshape(16, 512)

    def inv(shape):
        ndim = len(shape)
        return pl.BlockSpec(shape, lambda n, _nd=ndim: (0,) * _nd)

    out = pl.pallas_call(
        _net_kernel,
        out_shape=jax.ShapeDtypeStruct((N, fo, fo), jnp.float32),
        grid=(N,),
        in_specs=[
            pl.BlockSpec((1, C, H, W), lambda n: (n, 0, 0, 0)),
            pl.BlockSpec((1, C, H, W), lambda n: (n, 0, 0, 0)),
            inv(w0.shape), inv((1, 64)),
            inv(w1.shape), inv((1, 128)),
            inv(w2.shape), inv((1, 256)),
            inv(w3.shape), inv((1, 512)),
            inv(w4.shape), inv((1, 1)),
        ],
        out_specs=pl.BlockSpec((1, fo, fo), lambda n: (n, 0, 0)),
        scratch_shapes=[
            pltpu.VMEM((H + 2, W + 2, 128), jnp.float32),
            pltpu.VMEM((H // 2 + 2, W // 2 + 2, 128), jnp.float32),
            pltpu.VMEM((H // 4 + 2, W // 4 + 2, 128), jnp.float32),
            pltpu.VMEM((H // 8 + 2, W // 8 + 2, 128), jnp.float32),
            pltpu.VMEM((H // 8 + 2, W // 8 + 2, 128), jnp.float32),
        ],
        compiler_params=pltpu.CompilerParams(dimension_semantics=("parallel",)),
    )(raw, beautified, w0, b0.reshape(1, 64), w1, b1.reshape(1, 128),
      w2, b2.reshape(1, 256), w3, b3.reshape(1, 512),
      w4, b4.reshape(1, 1))
    return out[:, None, :, :]
